# baseline (speedup 1.0000x reference)
import functools
import jax
import jax.numpy as jnp
from jax import lax
from jax.experimental import pallas as pl
from jax.experimental.pallas import tpu as pltpu
from jax.experimental.pallas import tpu_sc as plsc

_TOPK_POS = 8
_TOPK_NEG = 64
_TEMP = 0.07
_INVT = 1.0 / 0.07
_BISECT_ITERS = 16
_SC_ROWS = 1536
_NBUC = 128
_BOUND = 1.002
_SCALE = _NBUC / (2 * _BOUND)
_INV_SCALE = (2 * _BOUND) / _NBUC
_NW = 32
_LN2 = 0.6931471805599453


def _make_tc_body(B, BR):
    kp_f = float(_TOPK_POS)
    kn_f = float(_TOPK_NEG)

    def body(new_ref, oldt_ref, trow_ref, tcol_ref, loss_ref, nz_ref):
        i = pl.program_id(0)
        new = new_ref[...]
        oldt = oldt_ref[...]
        nn = new / jnp.maximum(
            jnp.sqrt(jnp.sum(new * new, axis=1, keepdims=True)), 1e-12)
        on = oldt / jnp.maximum(
            jnp.sqrt(jnp.sum(oldt * oldt, axis=0, keepdims=True)), 1e-12)
        sim = jnp.dot(nn, on, preferred_element_type=jnp.float32)

        pm = trow_ref[...] == tcol_ref[...]
        spos = jnp.where(pm, sim, 2.0)
        sneg = jnp.where(pm, -2.0, sim)
        n_pos = jnp.sum(jnp.where(pm, 1.0, 0.0), axis=1, keepdims=True)
        k_p = jnp.minimum(n_pos, kp_f)
        k_n = jnp.minimum(float(B) - n_pos, kn_f)

        lo0 = jnp.full((BR, 1), -1.002, jnp.float32)
        hi0 = jnp.full((BR, 1), 1.002, jnp.float32)

        def it(_, c):
            lo_p, hi_p, lo_n, hi_n = c
            mid_p = 0.5 * (lo_p + hi_p)
            mid_n = 0.5 * (lo_n + hi_n)
            c_le = jnp.sum(jnp.where(spos <= mid_p, 1.0, 0.0),
                           axis=1, keepdims=True)
            c_ge = jnp.sum(jnp.where(sneg >= mid_n, 1.0, 0.0),
                           axis=1, keepdims=True)
            ok_p = c_le >= k_p
            hi_p = jnp.where(ok_p, mid_p, hi_p)
            lo_p = jnp.where(ok_p, lo_p, mid_p)
            ok_n = c_ge >= k_n
            lo_n = jnp.where(ok_n, mid_n, lo_n)
            hi_n = jnp.where(ok_n, hi_n, mid_n)
            return lo_p, hi_p, lo_n, hi_n

        lo_p, hi_p, lo_n, hi_n = jax.lax.fori_loop(
            0, _BISECT_ITERS, it, (lo0, hi0, lo0, hi0))

        e = jnp.exp((sim - 1.0) / _TEMP)
        selp = spos <= hi_p
        cnt_p = jnp.sum(jnp.where(selp, 1.0, 0.0), axis=1, keepdims=True)
        xs_p = cnt_p - k_p
        s_pos = (jnp.sum(jnp.where(selp, sim, 0.0), axis=1, keepdims=True)
                 - xs_p * hi_p)
        e_pos = (jnp.sum(jnp.where(selp, e, 0.0), axis=1, keepdims=True)
                 - xs_p * jnp.exp((hi_p - 1.0) / _TEMP))
        seln = sneg >= lo_n
        cnt_n = jnp.sum(jnp.where(seln, 1.0, 0.0), axis=1, keepdims=True)
        e_neg = (jnp.sum(jnp.where(seln, e, 0.0), axis=1, keepdims=True)
                 - (cnt_n - k_n) * jnp.exp((lo_n - 1.0) / _TEMP))

        lse = 1.0 / _TEMP + jnp.log(jnp.maximum(e_pos + e_neg, 1e-37))
        loss_rows = k_p * lse - s_pos / _TEMP
        nz_rows = jnp.where(loss_rows != 0.0, 1.0, 0.0)
        part_loss = jnp.sum(loss_rows, axis=0, keepdims=True)
        part_nz = jnp.sum(nz_rows, axis=0, keepdims=True)

        @pl.when(i == 0)
        def _():
            loss_ref[...] = part_loss
            nz_ref[...] = part_nz

        @pl.when(i != 0)
        def _():
            loss_ref[...] = loss_ref[...] + part_loss
            nz_ref[...] = nz_ref[...] + part_nz

    return body


def _tc_fused(old_feat, new_feat, trow, tcol, n_rows):
    B, D = old_feat.shape
    BR = 512
    return pl.pallas_call(
        _make_tc_body(B, BR),
        grid=(n_rows // BR,),
        in_specs=[
            pl.BlockSpec((BR, D), lambda i: (i, 0)),
            pl.BlockSpec((D, B), lambda i: (0, 0)),
            pl.BlockSpec((BR, 1), lambda i: (i, 0)),
            pl.BlockSpec((1, B), lambda i: (0, 0)),
        ],
        out_specs=[
            pl.BlockSpec((1, 1), lambda i: (0, 0)),
            pl.BlockSpec((1, 1), lambda i: (0, 0)),
        ],
        out_shape=[
            jax.ShapeDtypeStruct((1, 1), jnp.float32),
            jax.ShapeDtypeStruct((1, 1), jnp.float32),
        ],
        compiler_params=pltpu.CompilerParams(
            dimension_semantics=("arbitrary",)),
    )(new_feat, old_feat.T, trow, tcol)


def _tc_sim_body(new_ref, oldt_ref, sim_ref):
    new = new_ref[...]
    oldt = oldt_ref[...]
    nn = new / jnp.maximum(
        jnp.sqrt(jnp.sum(new * new, axis=1, keepdims=True)), 1e-12)
    on = oldt / jnp.maximum(
        jnp.sqrt(jnp.sum(oldt * oldt, axis=0, keepdims=True)), 1e-12)
    sim_ref[...] = jnp.dot(nn, on, preferred_element_type=jnp.float32)


def _tc_sim(old_feat, new_tail):
    B, D = old_feat.shape
    n_rows = new_tail.shape[0]
    BR = 512
    return pl.pallas_call(
        _tc_sim_body,
        grid=(n_rows // BR,),
        in_specs=[
            pl.BlockSpec((BR, D), lambda i: (i, 0)),
            pl.BlockSpec((D, B), lambda i: (0, 0)),
        ],
        out_specs=pl.BlockSpec((BR, B), lambda i: (i, 0)),
        out_shape=jax.ShapeDtypeStruct((n_rows, B), jnp.float32),
        compiler_params=pltpu.CompilerParams(
            dimension_semantics=("arbitrary",)),
    )(new_tail, old_feat.T)


def _vln(x):
    bits = plsc.bitcast(x, jnp.int32)
    ex = ((bits >> 23) & 0xFF) - 127
    m = plsc.bitcast((bits & 0x7FFFFF) | 0x3F800000, jnp.float32)
    t = (m - 1.0) / (m + 1.0)
    t2 = t * t
    ln_m = 2.0 * t * (1.0 + t2 * (1.0 / 3.0 + t2 * (0.2 + t2 * (1.0 / 7.0 + t2 / 9.0))))
    return ex.astype(jnp.float32) * _LN2 + ln_m


def _lane0(v):
    iota = lax.iota(jnp.int32, 16)
    return jnp.sum(jnp.where(iota == 0, v + iota * 0, 0 * v))


def _make_sc(B, sc_rows, row_off):
    rows_per_tile = sc_rows // _NW
    nvec4 = B // 64
    hc_words = 2 * _NBUC * 16
    mesh = plsc.VectorSubcoreMesh(core_axis_name="c", subcore_axis_name="s")

    @functools.partial(
        pl.kernel,
        mesh=mesh,
        out_type=jax.ShapeDtypeStruct((_NW, 16), jnp.float32),
        scratch_types=[
            pltpu.VMEM((B,), jnp.int32),
            pltpu.VMEM((B,), jnp.float32),
            pltpu.VMEM((B,), jnp.float32),
            pltpu.VMEM((hc_words,), jnp.float32),
            pltpu.VMEM((256,), jnp.float32),
            pltpu.VMEM((16,), jnp.float32),
            pltpu.SemaphoreType.DMA,
            pltpu.SemaphoreType.DMA,
        ],
        compiler_params=pltpu.CompilerParams(needs_layout_passes=False),
    )
    def sc_sel(sim_hbm, tcol_hbm, out_hbm, tcol_v, row0_v, row1_v, hc, hg,
               stage_v, sem0, sem1):
        wid = lax.axis_index("s") * 2 + lax.axis_index("c")
        base_row = wid * rows_per_tile
        pltpu.sync_copy(tcol_hbm, tcol_v)
        iota = lax.iota(jnp.int32, 16)
        zero16 = jnp.zeros((16,), jnp.float32)
        one16 = jnp.ones((16,), jnp.float32)

        def zh(j, _):
            base = pl.multiple_of(j * 64, 8)
            for u in range(4):
                hc[pl.ds(base + u * 16, 16)] = zero16
            return 0
        lax.fori_loop(0, hc_words // 64, zh, 0)
        for u in range(16):
            hg[pl.ds(u * 16, 16)] = zero16

        pltpu.async_copy(sim_hbm.at[base_row], row0_v, sem0)
        pltpu.async_copy(sim_hbm.at[base_row + 1], row1_v, sem1)

        def process_row(buf, row_abs):
            trow = plsc.load_gather(
                tcol_v, [jnp.full((16,), row_off + row_abs, jnp.int32)])

            def p1(j, _):
                base = pl.multiple_of(j * 64, 8)
                for u in range(4):
                    off = base + u * 16
                    s = buf[pl.ds(off, 16)]
                    tc = tcol_v[pl.ds(off, 16)]
                    pm = tc == trow
                    q = jnp.clip((s + _BOUND) * _SCALE, 0.0,
                                 float(_NBUC - 1)).astype(jnp.int32)
                    qq = jnp.where(pm, q, q + _NBUC)
                    plsc.addupdate_scatter(hc, [qq * 16 + iota], one16)
                    plsc.addupdate_scatter(hg, [(qq >> 4) * 16 + iota], one16)
                return 0
            lax.fori_loop(0, nvec4, p1, 0)

            gc = zero16
            for g in range(16):
                sg = jnp.sum(hg[pl.ds(g * 16, 16)])
                gc = jnp.where(iota == g, sg, gc)
            npos = jnp.sum(jnp.where(iota < 8, gc, 0.0))
            nneg = jnp.sum(jnp.where(iota >= 8, gc, 0.0))

            gpos_v = jnp.where(iota < 8, gc, 0.0)
            cumg_p = plsc.cumsum(gpos_v)
            crg_p = cumg_p >= float(_TOPK_POS)
            dp = jnp.sum(jnp.where(crg_p, 1, 0)) > 0
            gsp = jnp.minimum(_lane0(plsc.all_reduce_ffs(crg_p)), 7)
            cbg_p = jnp.sum(jnp.where(crg_p, zero16, gpos_v))
            bc_p = zero16
            for t in range(16):
                sb = jnp.sum(hc[pl.ds(pl.multiple_of(gsp * 256, 8) + t * 16,
                                      16)])
                bc_p = jnp.where(iota == t, sb, bc_p)
            lcum_p = plsc.cumsum(bc_p) + cbg_p
            crp = lcum_p >= float(_TOPK_POS)
            lp = _lane0(plsc.all_reduce_ffs(crp))
            bp = gsp * 16 + lp
            cbp = cbg_p + jnp.sum(jnp.where(crp, zero16, bc_p))

            rgc = lax.rev(gc, (0,))
            gneg_r = jnp.where(iota < 8, rgc, 0.0)
            cumg_n = plsc.cumsum(gneg_r)
            crg_n = cumg_n >= float(_TOPK_NEG)
            dn = jnp.sum(jnp.where(crg_n, 1, 0)) > 0
            gsn = jnp.clip(15 - _lane0(plsc.all_reduce_ffs(crg_n)), 8, 15)
            cag_n = jnp.sum(jnp.where(crg_n, zero16, gneg_r))
            bc_n = zero16
            for t in range(16):
                sb = jnp.sum(hc[pl.ds(pl.multiple_of(gsn * 256, 8) + t * 16,
                                      16)])
                bc_n = jnp.where(iota == t, sb, bc_n)
            rbc_n = lax.rev(bc_n, (0,))
            lcum_n = plsc.cumsum(rbc_n) + cag_n
            crn = lcum_n >= float(_TOPK_NEG)
            lu = _lane0(plsc.all_reduce_ffs(crn))
            bn = (gsn - 8) * 16 + (15 - lu)
            can = cag_n + jnp.sum(jnp.where(crn, zero16, rbc_n))

            k_p = jnp.where(dp, float(_TOPK_POS), npos)
            k_n = jnp.where(dn, float(_TOPK_NEG), nneg)
            bp_eff = jnp.where(dp, bp, _NBUC)
            bn_eff = jnp.where(dn, bn, -1)

            def zh2(j, _):
                base = pl.multiple_of(j * 64, 8)
                for u in range(4):
                    hc[pl.ds(base + u * 16, 16)] = zero16
                return 0
            lax.fori_loop(0, hc_words // 64, zh2, 0)
            for u in range(16):
                hg[pl.ds(u * 16, 16)] = zero16

            def p2(j, c):
                sacc, eaccp, eaccn = c
                base = pl.multiple_of(j * 64, 8)
                for u in range(4):
                    off = base + u * 16
                    s = buf[pl.ds(off, 16)]
                    tc = tcol_v[pl.ds(off, 16)]
                    pm = tc == trow
                    q = jnp.clip((s + _BOUND) * _SCALE, 0.0,
                                 float(_NBUC - 1)).astype(jnp.int32)
                    e = jnp.exp((s - 1.0) * _INVT)
                    selp = jnp.logical_and(pm, q < bp_eff)
                    seln = jnp.logical_and(jnp.logical_not(pm), q > bn_eff)
                    sacc = sacc + jnp.where(selp, s, 0.0)
                    eaccp = eaccp + jnp.where(selp, e, 0.0)
                    eaccn = eaccn + jnp.where(seln, e, 0.0)
                return sacc, eaccp, eaccn

            sacc, eaccp, eaccn = lax.fori_loop(
                0, nvec4, p2, (zero16, zero16, zero16))
            s_below = jnp.sum(sacc)
            e_below = jnp.sum(eaccp)
            e_above = jnp.sum(eaccn)

            mid_p = (bp_eff.astype(jnp.float32) + 0.5) * _INV_SCALE - _BOUND
            mid_n = (bn_eff.astype(jnp.float32) + 0.5) * _INV_SCALE - _BOUND
            r_p = jnp.where(dp, k_p - cbp, 0.0)
            r_n = jnp.where(dn, k_n - can, 0.0)
            ep_mid = _lane0(jnp.exp(jnp.full((16,), (mid_p - 1.0) * _INVT)))
            en_mid = _lane0(jnp.exp(jnp.full((16,), (mid_n - 1.0) * _INVT)))
            s_pos = s_below + r_p * mid_p
            e_all = jnp.maximum(
                e_below + r_p * ep_mid + e_above + r_n * en_mid, 1e-37)
            ln_e = _lane0(_vln(jnp.full((16,), e_all)))
            loss_row = k_p * (_INVT + ln_e) - s_pos * _INVT
            nz_row = jnp.where(loss_row != 0.0, 1.0, 0.0)
            return loss_row, nz_row

        def pair_body(i, acc):
            loss_acc, nz_acc = acc
            r0 = base_row + 2 * i
            pltpu.make_async_copy(sim_hbm.at[r0], row0_v, sem0).wait()
            l0, n0 = process_row(row0_v, r0)

            @pl.when(i < rows_per_tile // 2 - 1)
            def _():
                pltpu.async_copy(sim_hbm.at[r0 + 2], row0_v, sem0)

            pltpu.make_async_copy(sim_hbm.at[r0 + 1], row1_v, sem1).wait()
            l1, n1 = process_row(row1_v, r0 + 1)

            @pl.when(i < rows_per_tile // 2 - 1)
            def _():
                pltpu.async_copy(sim_hbm.at[r0 + 3], row1_v, sem1)

            return loss_acc + l0 + l1, nz_acc + n0 + n1

        loss_sum, nz_sum = lax.fori_loop(0, rows_per_tile // 2, pair_body,
                                         (0.0, 0.0))
        out_vec = jnp.where(iota == 0, loss_sum,
                            jnp.where(iota == 1, nz_sum, 0.0))
        stage_v[...] = out_vec
        pltpu.sync_copy(stage_v, out_hbm.at[wid])

    return sc_sel


def kernel(old_feat, new_feat, target):
    B, D = old_feat.shape
    tc_rows = B - _SC_ROWS
    tgt = target.astype(jnp.int32)
    trow = tgt.astype(jnp.float32).reshape(B, 1)
    tcol = tgt.astype(jnp.float32).reshape(1, B)
    sim_tail = _tc_sim(old_feat, new_feat[tc_rows:])
    parts_sc = _make_sc(B, _SC_ROWS, tc_rows)(sim_tail, tgt)
    tc_loss, tc_nz = _tc_fused(old_feat, new_feat, trow, tcol, tc_rows)
    loss = tc_loss[0, 0] + jnp.sum(parts_sc[:, 0])
    nz = tc_nz[0, 0] + jnp.sum(parts_sc[:, 1])
    return loss / jnp.maximum(nz, 1.0)


# speedup vs baseline: 1.1346x; 1.1346x over previous
import functools
import jax
import jax.numpy as jnp
from jax import lax
from jax.experimental import pallas as pl
from jax.experimental.pallas import tpu as pltpu
from jax.experimental.pallas import tpu_sc as plsc

_TOPK_POS = 8
_TOPK_NEG = 64
_TEMP = 0.07
_INVT = 1.0 / 0.07
_BISECT_ITERS = 16
_SC_ROWS = 1280
_NBUC = 128
_BOUND = 1.002
_SCALE = _NBUC / (2 * _BOUND)
_INV_SCALE = (2 * _BOUND) / _NBUC
_NW = 32
_LN2 = 0.6931471805599453


def _make_tc_body(B, BR):
    kp_f = float(_TOPK_POS)
    kn_f = float(_TOPK_NEG)

    def body(new_ref, oldt_ref, trow_ref, tcol_ref, loss_ref, nz_ref):
        i = pl.program_id(0)
        new = new_ref[...]
        oldt = oldt_ref[...]
        nn = new / jnp.maximum(
            jnp.sqrt(jnp.sum(new * new, axis=1, keepdims=True)), 1e-12)
        on = oldt / jnp.maximum(
            jnp.sqrt(jnp.sum(oldt * oldt, axis=0, keepdims=True)), 1e-12)
        sim = jnp.dot(nn, on, preferred_element_type=jnp.float32)

        pm = trow_ref[...] == tcol_ref[...]
        spos = jnp.where(pm, sim, 2.0)
        sneg = jnp.where(pm, -2.0, sim)
        n_pos = jnp.sum(jnp.where(pm, 1.0, 0.0), axis=1, keepdims=True)
        k_p = jnp.minimum(n_pos, kp_f)
        k_n = jnp.minimum(float(B) - n_pos, kn_f)

        lo0 = jnp.full((BR, 1), -1.002, jnp.float32)
        hi0 = jnp.full((BR, 1), 1.002, jnp.float32)

        def it(_, c):
            lo_p, hi_p, lo_n, hi_n = c
            mid_p = 0.5 * (lo_p + hi_p)
            mid_n = 0.5 * (lo_n + hi_n)
            c_le = jnp.sum(jnp.where(spos <= mid_p, 1.0, 0.0),
                           axis=1, keepdims=True)
            c_ge = jnp.sum(jnp.where(sneg >= mid_n, 1.0, 0.0),
                           axis=1, keepdims=True)
            ok_p = c_le >= k_p
            hi_p = jnp.where(ok_p, mid_p, hi_p)
            lo_p = jnp.where(ok_p, lo_p, mid_p)
            ok_n = c_ge >= k_n
            lo_n = jnp.where(ok_n, mid_n, lo_n)
            hi_n = jnp.where(ok_n, hi_n, mid_n)
            return lo_p, hi_p, lo_n, hi_n

        lo_p, hi_p, lo_n, hi_n = jax.lax.fori_loop(
            0, _BISECT_ITERS, it, (lo0, hi0, lo0, hi0))

        e = jnp.exp((sim - 1.0) / _TEMP)
        selp = spos <= hi_p
        cnt_p = jnp.sum(jnp.where(selp, 1.0, 0.0), axis=1, keepdims=True)
        xs_p = cnt_p - k_p
        s_pos = (jnp.sum(jnp.where(selp, sim, 0.0), axis=1, keepdims=True)
                 - xs_p * hi_p)
        e_pos = (jnp.sum(jnp.where(selp, e, 0.0), axis=1, keepdims=True)
                 - xs_p * jnp.exp((hi_p - 1.0) / _TEMP))
        seln = sneg >= lo_n
        cnt_n = jnp.sum(jnp.where(seln, 1.0, 0.0), axis=1, keepdims=True)
        e_neg = (jnp.sum(jnp.where(seln, e, 0.0), axis=1, keepdims=True)
                 - (cnt_n - k_n) * jnp.exp((lo_n - 1.0) / _TEMP))

        lse = 1.0 / _TEMP + jnp.log(jnp.maximum(e_pos + e_neg, 1e-37))
        loss_rows = k_p * lse - s_pos / _TEMP
        nz_rows = jnp.where(loss_rows != 0.0, 1.0, 0.0)
        part_loss = jnp.sum(loss_rows, axis=0, keepdims=True)
        part_nz = jnp.sum(nz_rows, axis=0, keepdims=True)

        @pl.when(i == 0)
        def _():
            loss_ref[...] = part_loss
            nz_ref[...] = part_nz

        @pl.when(i != 0)
        def _():
            loss_ref[...] = loss_ref[...] + part_loss
            nz_ref[...] = nz_ref[...] + part_nz

    return body


def _tc_fused(old_feat, new_feat, trow, tcol, n_rows):
    B, D = old_feat.shape
    BR = 256
    return pl.pallas_call(
        _make_tc_body(B, BR),
        grid=(n_rows // BR,),
        in_specs=[
            pl.BlockSpec((BR, D), lambda i: (i, 0)),
            pl.BlockSpec((D, B), lambda i: (0, 0)),
            pl.BlockSpec((BR, 1), lambda i: (i, 0)),
            pl.BlockSpec((1, B), lambda i: (0, 0)),
        ],
        out_specs=[
            pl.BlockSpec((1, 1), lambda i: (0, 0)),
            pl.BlockSpec((1, 1), lambda i: (0, 0)),
        ],
        out_shape=[
            jax.ShapeDtypeStruct((1, 1), jnp.float32),
            jax.ShapeDtypeStruct((1, 1), jnp.float32),
        ],
        compiler_params=pltpu.CompilerParams(
            dimension_semantics=("arbitrary",)),
    )(new_feat, old_feat.T, trow, tcol)


def _tc_sim_body(new_ref, oldt_ref, sim_ref):
    new = new_ref[...]
    oldt = oldt_ref[...]
    nn = new / jnp.maximum(
        jnp.sqrt(jnp.sum(new * new, axis=1, keepdims=True)), 1e-12)
    on = oldt / jnp.maximum(
        jnp.sqrt(jnp.sum(oldt * oldt, axis=0, keepdims=True)), 1e-12)
    sim_ref[...] = jnp.dot(nn, on, preferred_element_type=jnp.float32)


def _tc_sim(old_feat, new_tail):
    B, D = old_feat.shape
    n_rows = new_tail.shape[0]
    BR = 256
    return pl.pallas_call(
        _tc_sim_body,
        grid=(n_rows // BR,),
        in_specs=[
            pl.BlockSpec((BR, D), lambda i: (i, 0)),
            pl.BlockSpec((D, B), lambda i: (0, 0)),
        ],
        out_specs=pl.BlockSpec((BR, B), lambda i: (i, 0)),
        out_shape=jax.ShapeDtypeStruct((n_rows, B), jnp.float32),
        compiler_params=pltpu.CompilerParams(
            dimension_semantics=("arbitrary",)),
    )(new_tail, old_feat.T)


def _vln(x):
    bits = plsc.bitcast(x, jnp.int32)
    ex = ((bits >> 23) & 0xFF) - 127
    m = plsc.bitcast((bits & 0x7FFFFF) | 0x3F800000, jnp.float32)
    t = (m - 1.0) / (m + 1.0)
    t2 = t * t
    ln_m = 2.0 * t * (1.0 + t2 * (1.0 / 3.0 + t2 * (0.2 + t2 * (1.0 / 7.0 + t2 / 9.0))))
    return ex.astype(jnp.float32) * _LN2 + ln_m


def _lane0(v):
    iota = lax.iota(jnp.int32, 16)
    return jnp.sum(jnp.where(iota == 0, v + iota * 0, 0 * v))


def _make_sc(B, sc_rows, row_off):
    rows_per_tile = sc_rows // _NW
    nvec4 = B // 64
    hc_words = 2 * _NBUC * 16
    mesh = plsc.VectorSubcoreMesh(core_axis_name="c", subcore_axis_name="s")

    @functools.partial(
        pl.kernel,
        mesh=mesh,
        out_type=jax.ShapeDtypeStruct((_NW, 16), jnp.float32),
        scratch_types=[
            pltpu.VMEM((B,), jnp.int32),
            pltpu.VMEM((B,), jnp.float32),
            pltpu.VMEM((B,), jnp.float32),
            pltpu.VMEM((hc_words,), jnp.float32),
            pltpu.VMEM((256,), jnp.float32),
            pltpu.VMEM((16,), jnp.float32),
            pltpu.SemaphoreType.DMA,
            pltpu.SemaphoreType.DMA,
        ],
        compiler_params=pltpu.CompilerParams(needs_layout_passes=False),
    )
    def sc_sel(sim_hbm, tcol_hbm, out_hbm, tcol_v, row0_v, row1_v, hc, hg,
               stage_v, sem0, sem1):
        wid = lax.axis_index("s") * 2 + lax.axis_index("c")
        base_row = wid * rows_per_tile
        pltpu.sync_copy(tcol_hbm, tcol_v)
        iota = lax.iota(jnp.int32, 16)
        zero16 = jnp.zeros((16,), jnp.float32)
        one16 = jnp.ones((16,), jnp.float32)

        def zh(j, _):
            base = pl.multiple_of(j * 64, 8)
            for u in range(4):
                hc[pl.ds(base + u * 16, 16)] = zero16
            return 0
        lax.fori_loop(0, hc_words // 64, zh, 0)
        for u in range(16):
            hg[pl.ds(u * 16, 16)] = zero16

        pltpu.async_copy(sim_hbm.at[base_row], row0_v, sem0)
        pltpu.async_copy(sim_hbm.at[base_row + 1], row1_v, sem1)

        def process_row(buf, row_abs):
            trow = plsc.load_gather(
                tcol_v, [jnp.full((16,), row_off + row_abs, jnp.int32)])

            def p1(j, _):
                base = pl.multiple_of(j * 64, 8)
                for u in range(4):
                    off = base + u * 16
                    s = buf[pl.ds(off, 16)]
                    tc = tcol_v[pl.ds(off, 16)]
                    pm = tc == trow
                    q = jnp.clip((s + _BOUND) * _SCALE, 0.0,
                                 float(_NBUC - 1)).astype(jnp.int32)
                    qq = jnp.where(pm, q, q + _NBUC)
                    plsc.addupdate_scatter(hc, [qq * 16 + iota], one16)
                    plsc.addupdate_scatter(hg, [(qq >> 4) * 16 + iota], one16)
                return 0
            lax.fori_loop(0, nvec4, p1, 0)

            gc = zero16
            for g in range(16):
                sg = jnp.sum(hg[pl.ds(g * 16, 16)])
                gc = jnp.where(iota == g, sg, gc)
            npos = jnp.sum(jnp.where(iota < 8, gc, 0.0))
            nneg = jnp.sum(jnp.where(iota >= 8, gc, 0.0))

            gpos_v = jnp.where(iota < 8, gc, 0.0)
            cumg_p = plsc.cumsum(gpos_v)
            crg_p = cumg_p >= float(_TOPK_POS)
            dp = jnp.sum(jnp.where(crg_p, 1, 0)) > 0
            gsp = jnp.minimum(_lane0(plsc.all_reduce_ffs(crg_p)), 7)
            cbg_p = jnp.sum(jnp.where(crg_p, zero16, gpos_v))
            bc_p = zero16
            for t in range(16):
                sb = jnp.sum(hc[pl.ds(pl.multiple_of(gsp * 256, 8) + t * 16,
                                      16)])
                bc_p = jnp.where(iota == t, sb, bc_p)
            lcum_p = plsc.cumsum(bc_p) + cbg_p
            crp = lcum_p >= float(_TOPK_POS)
            lp = _lane0(plsc.all_reduce_ffs(crp))
            bp = gsp * 16 + lp
            cbp = cbg_p + jnp.sum(jnp.where(crp, zero16, bc_p))

            rgc = lax.rev(gc, (0,))
            gneg_r = jnp.where(iota < 8, rgc, 0.0)
            cumg_n = plsc.cumsum(gneg_r)
            crg_n = cumg_n >= float(_TOPK_NEG)
            dn = jnp.sum(jnp.where(crg_n, 1, 0)) > 0
            gsn = jnp.clip(15 - _lane0(plsc.all_reduce_ffs(crg_n)), 8, 15)
            cag_n = jnp.sum(jnp.where(crg_n, zero16, gneg_r))
            bc_n = zero16
            for t in range(16):
                sb = jnp.sum(hc[pl.ds(pl.multiple_of(gsn * 256, 8) + t * 16,
                                      16)])
                bc_n = jnp.where(iota == t, sb, bc_n)
            rbc_n = lax.rev(bc_n, (0,))
            lcum_n = plsc.cumsum(rbc_n) + cag_n
            crn = lcum_n >= float(_TOPK_NEG)
            lu = _lane0(plsc.all_reduce_ffs(crn))
            bn = (gsn - 8) * 16 + (15 - lu)
            can = cag_n + jnp.sum(jnp.where(crn, zero16, rbc_n))

            k_p = jnp.where(dp, float(_TOPK_POS), npos)
            k_n = jnp.where(dn, float(_TOPK_NEG), nneg)
            bp_eff = jnp.where(dp, bp, _NBUC)
            bn_eff = jnp.where(dn, bn, -1)

            def zh2(j, _):
                base = pl.multiple_of(j * 64, 8)
                for u in range(4):
                    hc[pl.ds(base + u * 16, 16)] = zero16
                return 0
            lax.fori_loop(0, hc_words // 64, zh2, 0)
            for u in range(16):
                hg[pl.ds(u * 16, 16)] = zero16

            def p2(j, c):
                sacc, eaccp, eaccn = c
                base = pl.multiple_of(j * 64, 8)
                for u in range(4):
                    off = base + u * 16
                    s = buf[pl.ds(off, 16)]
                    tc = tcol_v[pl.ds(off, 16)]
                    pm = tc == trow
                    q = jnp.clip((s + _BOUND) * _SCALE, 0.0,
                                 float(_NBUC - 1)).astype(jnp.int32)
                    e = jnp.exp((s - 1.0) * _INVT)
                    selp = jnp.logical_and(pm, q < bp_eff)
                    seln = jnp.logical_and(jnp.logical_not(pm), q > bn_eff)
                    sacc = sacc + jnp.where(selp, s, 0.0)
                    eaccp = eaccp + jnp.where(selp, e, 0.0)
                    eaccn = eaccn + jnp.where(seln, e, 0.0)
                return sacc, eaccp, eaccn

            sacc, eaccp, eaccn = lax.fori_loop(
                0, nvec4, p2, (zero16, zero16, zero16))
            s_below = jnp.sum(sacc)
            e_below = jnp.sum(eaccp)
            e_above = jnp.sum(eaccn)

            mid_p = (bp_eff.astype(jnp.float32) + 0.5) * _INV_SCALE - _BOUND
            mid_n = (bn_eff.astype(jnp.float32) + 0.5) * _INV_SCALE - _BOUND
            r_p = jnp.where(dp, k_p - cbp, 0.0)
            r_n = jnp.where(dn, k_n - can, 0.0)
            ep_mid = _lane0(jnp.exp(jnp.full((16,), (mid_p - 1.0) * _INVT)))
            en_mid = _lane0(jnp.exp(jnp.full((16,), (mid_n - 1.0) * _INVT)))
            s_pos = s_below + r_p * mid_p
            e_all = jnp.maximum(
                e_below + r_p * ep_mid + e_above + r_n * en_mid, 1e-37)
            ln_e = _lane0(_vln(jnp.full((16,), e_all)))
            loss_row = k_p * (_INVT + ln_e) - s_pos * _INVT
            nz_row = jnp.where(loss_row != 0.0, 1.0, 0.0)
            return loss_row, nz_row

        def pair_body(i, acc):
            loss_acc, nz_acc = acc
            r0 = base_row + 2 * i
            pltpu.make_async_copy(sim_hbm.at[r0], row0_v, sem0).wait()
            l0, n0 = process_row(row0_v, r0)

            @pl.when(i < rows_per_tile // 2 - 1)
            def _():
                pltpu.async_copy(sim_hbm.at[r0 + 2], row0_v, sem0)

            pltpu.make_async_copy(sim_hbm.at[r0 + 1], row1_v, sem1).wait()
            l1, n1 = process_row(row1_v, r0 + 1)

            @pl.when(i < rows_per_tile // 2 - 1)
            def _():
                pltpu.async_copy(sim_hbm.at[r0 + 3], row1_v, sem1)

            return loss_acc + l0 + l1, nz_acc + n0 + n1

        loss_sum, nz_sum = lax.fori_loop(0, rows_per_tile // 2, pair_body,
                                         (0.0, 0.0))
        out_vec = jnp.where(iota == 0, loss_sum,
                            jnp.where(iota == 1, nz_sum, 0.0))
        stage_v[...] = out_vec
        pltpu.sync_copy(stage_v, out_hbm.at[wid])

    return sc_sel


def kernel(old_feat, new_feat, target):
    B, D = old_feat.shape
    tc_rows = B - _SC_ROWS
    tgt = target.astype(jnp.int32)
    trow = tgt.astype(jnp.float32).reshape(B, 1)
    tcol = tgt.astype(jnp.float32).reshape(1, B)
    sim_tail = _tc_sim(old_feat, new_feat[tc_rows:])
    parts_sc = _make_sc(B, _SC_ROWS, tc_rows)(sim_tail, tgt)
    tc_loss, tc_nz = _tc_fused(old_feat, new_feat, trow, tcol, tc_rows)
    loss = tc_loss[0, 0] + jnp.sum(parts_sc[:, 0])
    nz = tc_nz[0, 0] + jnp.sum(parts_sc[:, 1])
    return loss / jnp.maximum(nz, 1.0)


# speedup vs baseline: 1.1716x; 1.0326x over previous
import functools
import jax
import jax.numpy as jnp
from jax import lax
from jax.experimental import pallas as pl
from jax.experimental.pallas import tpu as pltpu
from jax.experimental.pallas import tpu_sc as plsc

_TOPK_POS = 8
_TOPK_NEG = 64
_TEMP = 0.07
_INVT = 1.0 / 0.07
_BISECT_ITERS = 13
_SC_ROWS = 1280
_NBUC = 128
_BOUND = 1.002
_SCALE = _NBUC / (2 * _BOUND)
_INV_SCALE = (2 * _BOUND) / _NBUC
_NW = 32
_LN2 = 0.6931471805599453


def _make_tc_body(B, BR):
    kp_f = float(_TOPK_POS)
    kn_f = float(_TOPK_NEG)

    def body(new_ref, oldt_ref, trow_ref, tcol_ref, loss_ref, nz_ref):
        i = pl.program_id(0)
        new = new_ref[...]
        oldt = oldt_ref[...]
        nn = new / jnp.maximum(
            jnp.sqrt(jnp.sum(new * new, axis=1, keepdims=True)), 1e-12)
        on = oldt / jnp.maximum(
            jnp.sqrt(jnp.sum(oldt * oldt, axis=0, keepdims=True)), 1e-12)
        sim = jnp.dot(nn, on, preferred_element_type=jnp.float32)

        pm = trow_ref[...] == tcol_ref[...]
        spos = jnp.where(pm, sim, 2.0)
        sneg = jnp.where(pm, -2.0, sim)
        n_pos = jnp.sum(jnp.where(pm, 1.0, 0.0), axis=1, keepdims=True)
        k_p = jnp.minimum(n_pos, kp_f)
        k_n = jnp.minimum(float(B) - n_pos, kn_f)

        lo0 = jnp.full((BR, 1), -1.002, jnp.float32)
        hi0 = jnp.full((BR, 1), 1.002, jnp.float32)

        def it(_, c):
            lo_p, hi_p, lo_n, hi_n = c
            mid_p = 0.5 * (lo_p + hi_p)
            mid_n = 0.5 * (lo_n + hi_n)
            c_le = jnp.sum(jnp.where(spos <= mid_p, 1.0, 0.0),
                           axis=1, keepdims=True)
            c_ge = jnp.sum(jnp.where(sneg >= mid_n, 1.0, 0.0),
                           axis=1, keepdims=True)
            ok_p = c_le >= k_p
            hi_p = jnp.where(ok_p, mid_p, hi_p)
            lo_p = jnp.where(ok_p, lo_p, mid_p)
            ok_n = c_ge >= k_n
            lo_n = jnp.where(ok_n, mid_n, lo_n)
            hi_n = jnp.where(ok_n, hi_n, mid_n)
            return lo_p, hi_p, lo_n, hi_n

        lo_p, hi_p, lo_n, hi_n = jax.lax.fori_loop(
            0, _BISECT_ITERS, it, (lo0, hi0, lo0, hi0))

        e = jnp.exp((sim - 1.0) / _TEMP)
        selp = spos <= hi_p
        cnt_p = jnp.sum(jnp.where(selp, 1.0, 0.0), axis=1, keepdims=True)
        xs_p = cnt_p - k_p
        s_pos = (jnp.sum(jnp.where(selp, sim, 0.0), axis=1, keepdims=True)
                 - xs_p * hi_p)
        e_pos = (jnp.sum(jnp.where(selp, e, 0.0), axis=1, keepdims=True)
                 - xs_p * jnp.exp((hi_p - 1.0) / _TEMP))
        seln = sneg >= lo_n
        cnt_n = jnp.sum(jnp.where(seln, 1.0, 0.0), axis=1, keepdims=True)
        e_neg = (jnp.sum(jnp.where(seln, e, 0.0), axis=1, keepdims=True)
                 - (cnt_n - k_n) * jnp.exp((lo_n - 1.0) / _TEMP))

        lse = 1.0 / _TEMP + jnp.log(jnp.maximum(e_pos + e_neg, 1e-37))
        loss_rows = k_p * lse - s_pos / _TEMP
        nz_rows = jnp.where(loss_rows != 0.0, 1.0, 0.0)
        part_loss = jnp.sum(loss_rows, axis=0, keepdims=True)
        part_nz = jnp.sum(nz_rows, axis=0, keepdims=True)

        @pl.when(i == 0)
        def _():
            loss_ref[...] = part_loss
            nz_ref[...] = part_nz

        @pl.when(i != 0)
        def _():
            loss_ref[...] = loss_ref[...] + part_loss
            nz_ref[...] = nz_ref[...] + part_nz

    return body


def _tc_fused(old_feat, new_feat, trow, tcol, n_rows):
    B, D = old_feat.shape
    BR = 256
    return pl.pallas_call(
        _make_tc_body(B, BR),
        grid=(n_rows // BR,),
        in_specs=[
            pl.BlockSpec((BR, D), lambda i: (i, 0)),
            pl.BlockSpec((D, B), lambda i: (0, 0)),
            pl.BlockSpec((BR, 1), lambda i: (i, 0)),
            pl.BlockSpec((1, B), lambda i: (0, 0)),
        ],
        out_specs=[
            pl.BlockSpec((1, 1), lambda i: (0, 0)),
            pl.BlockSpec((1, 1), lambda i: (0, 0)),
        ],
        out_shape=[
            jax.ShapeDtypeStruct((1, 1), jnp.float32),
            jax.ShapeDtypeStruct((1, 1), jnp.float32),
        ],
        compiler_params=pltpu.CompilerParams(
            dimension_semantics=("arbitrary",)),
    )(new_feat, old_feat.T, trow, tcol)


def _tc_sim_body(new_ref, oldt_ref, sim_ref):
    new = new_ref[...]
    oldt = oldt_ref[...]
    nn = new / jnp.maximum(
        jnp.sqrt(jnp.sum(new * new, axis=1, keepdims=True)), 1e-12)
    on = oldt / jnp.maximum(
        jnp.sqrt(jnp.sum(oldt * oldt, axis=0, keepdims=True)), 1e-12)
    sim_ref[...] = jnp.dot(nn, on, preferred_element_type=jnp.float32)


def _tc_sim(old_feat, new_tail):
    B, D = old_feat.shape
    n_rows = new_tail.shape[0]
    BR = 256
    return pl.pallas_call(
        _tc_sim_body,
        grid=(n_rows // BR,),
        in_specs=[
            pl.BlockSpec((BR, D), lambda i: (i, 0)),
            pl.BlockSpec((D, B), lambda i: (0, 0)),
        ],
        out_specs=pl.BlockSpec((BR, B), lambda i: (i, 0)),
        out_shape=jax.ShapeDtypeStruct((n_rows, B), jnp.float32),
        compiler_params=pltpu.CompilerParams(
            dimension_semantics=("arbitrary",)),
    )(new_tail, old_feat.T)


def _vln(x):
    bits = plsc.bitcast(x, jnp.int32)
    ex = ((bits >> 23) & 0xFF) - 127
    m = plsc.bitcast((bits & 0x7FFFFF) | 0x3F800000, jnp.float32)
    t = (m - 1.0) / (m + 1.0)
    t2 = t * t
    ln_m = 2.0 * t * (1.0 + t2 * (1.0 / 3.0 + t2 * (0.2 + t2 * (1.0 / 7.0 + t2 / 9.0))))
    return ex.astype(jnp.float32) * _LN2 + ln_m


def _lane0(v):
    iota = lax.iota(jnp.int32, 16)
    return jnp.sum(jnp.where(iota == 0, v + iota * 0, 0 * v))


def _make_sc(B, sc_rows, row_off):
    rows_per_tile = sc_rows // _NW
    nvec4 = B // 64
    hc_words = 2 * _NBUC * 16
    mesh = plsc.VectorSubcoreMesh(core_axis_name="c", subcore_axis_name="s")

    @functools.partial(
        pl.kernel,
        mesh=mesh,
        out_type=jax.ShapeDtypeStruct((_NW, 16), jnp.float32),
        scratch_types=[
            pltpu.VMEM((B,), jnp.int32),
            pltpu.VMEM((B,), jnp.float32),
            pltpu.VMEM((B,), jnp.float32),
            pltpu.VMEM((hc_words,), jnp.float32),
            pltpu.VMEM((256,), jnp.float32),
            pltpu.VMEM((16,), jnp.float32),
            pltpu.SemaphoreType.DMA,
            pltpu.SemaphoreType.DMA,
        ],
        compiler_params=pltpu.CompilerParams(needs_layout_passes=False),
    )
    def sc_sel(sim_hbm, tcol_hbm, out_hbm, tcol_v, row0_v, row1_v, hc, hg,
               stage_v, sem0, sem1):
        wid = lax.axis_index("s") * 2 + lax.axis_index("c")
        base_row = wid * rows_per_tile
        pltpu.sync_copy(tcol_hbm, tcol_v)
        iota = lax.iota(jnp.int32, 16)
        zero16 = jnp.zeros((16,), jnp.float32)
        one16 = jnp.ones((16,), jnp.float32)

        def zh(j, _):
            base = pl.multiple_of(j * 64, 8)
            for u in range(4):
                hc[pl.ds(base + u * 16, 16)] = zero16
            return 0
        lax.fori_loop(0, hc_words // 64, zh, 0)
        for u in range(16):
            hg[pl.ds(u * 16, 16)] = zero16

        pltpu.async_copy(sim_hbm.at[base_row], row0_v, sem0)
        pltpu.async_copy(sim_hbm.at[base_row + 1], row1_v, sem1)

        def process_row(buf, row_abs):
            trow = plsc.load_gather(
                tcol_v, [jnp.full((16,), row_off + row_abs, jnp.int32)])

            def p1(j, _):
                base = pl.multiple_of(j * 64, 8)
                for u in range(4):
                    off = base + u * 16
                    s = buf[pl.ds(off, 16)]
                    tc = tcol_v[pl.ds(off, 16)]
                    pm = tc == trow
                    q = jnp.clip((s + _BOUND) * _SCALE, 0.0,
                                 float(_NBUC - 1)).astype(jnp.int32)
                    qq = jnp.where(pm, q, q + _NBUC)
                    plsc.addupdate_scatter(hc, [qq * 16 + iota], one16)
                    plsc.addupdate_scatter(hg, [(qq >> 4) * 16 + iota], one16)
                return 0
            lax.fori_loop(0, nvec4, p1, 0)

            gc = zero16
            for g in range(16):
                sg = jnp.sum(hg[pl.ds(g * 16, 16)])
                gc = jnp.where(iota == g, sg, gc)
            npos = jnp.sum(jnp.where(iota < 8, gc, 0.0))
            nneg = jnp.sum(jnp.where(iota >= 8, gc, 0.0))

            gpos_v = jnp.where(iota < 8, gc, 0.0)
            cumg_p = plsc.cumsum(gpos_v)
            crg_p = cumg_p >= float(_TOPK_POS)
            dp = jnp.sum(jnp.where(crg_p, 1, 0)) > 0
            gsp = jnp.minimum(_lane0(plsc.all_reduce_ffs(crg_p)), 7)
            cbg_p = jnp.sum(jnp.where(crg_p, zero16, gpos_v))
            bc_p = zero16
            for t in range(16):
                sb = jnp.sum(hc[pl.ds(pl.multiple_of(gsp * 256, 8) + t * 16,
                                      16)])
                bc_p = jnp.where(iota == t, sb, bc_p)
            lcum_p = plsc.cumsum(bc_p) + cbg_p
            crp = lcum_p >= float(_TOPK_POS)
            lp = _lane0(plsc.all_reduce_ffs(crp))
            bp = gsp * 16 + lp
            cbp = cbg_p + jnp.sum(jnp.where(crp, zero16, bc_p))

            rgc = lax.rev(gc, (0,))
            gneg_r = jnp.where(iota < 8, rgc, 0.0)
            cumg_n = plsc.cumsum(gneg_r)
            crg_n = cumg_n >= float(_TOPK_NEG)
            dn = jnp.sum(jnp.where(crg_n, 1, 0)) > 0
            gsn = jnp.clip(15 - _lane0(plsc.all_reduce_ffs(crg_n)), 8, 15)
            cag_n = jnp.sum(jnp.where(crg_n, zero16, gneg_r))
            bc_n = zero16
            for t in range(16):
                sb = jnp.sum(hc[pl.ds(pl.multiple_of(gsn * 256, 8) + t * 16,
                                      16)])
                bc_n = jnp.where(iota == t, sb, bc_n)
            rbc_n = lax.rev(bc_n, (0,))
            lcum_n = plsc.cumsum(rbc_n) + cag_n
            crn = lcum_n >= float(_TOPK_NEG)
            lu = _lane0(plsc.all_reduce_ffs(crn))
            bn = (gsn - 8) * 16 + (15 - lu)
            can = cag_n + jnp.sum(jnp.where(crn, zero16, rbc_n))

            k_p = jnp.where(dp, float(_TOPK_POS), npos)
            k_n = jnp.where(dn, float(_TOPK_NEG), nneg)
            bp_eff = jnp.where(dp, bp, _NBUC)
            bn_eff = jnp.where(dn, bn, -1)

            def zh2(j, _):
                base = pl.multiple_of(j * 64, 8)
                for u in range(4):
                    hc[pl.ds(base + u * 16, 16)] = zero16
                return 0
            lax.fori_loop(0, hc_words // 64, zh2, 0)
            for u in range(16):
                hg[pl.ds(u * 16, 16)] = zero16

            def p2(j, c):
                sacc, eaccp, eaccn = c
                base = pl.multiple_of(j * 64, 8)
                for u in range(4):
                    off = base + u * 16
                    s = buf[pl.ds(off, 16)]
                    tc = tcol_v[pl.ds(off, 16)]
                    pm = tc == trow
                    q = jnp.clip((s + _BOUND) * _SCALE, 0.0,
                                 float(_NBUC - 1)).astype(jnp.int32)
                    e = jnp.exp((s - 1.0) * _INVT)
                    selp = jnp.logical_and(pm, q < bp_eff)
                    seln = jnp.logical_and(jnp.logical_not(pm), q > bn_eff)
                    sacc = sacc + jnp.where(selp, s, 0.0)
                    eaccp = eaccp + jnp.where(selp, e, 0.0)
                    eaccn = eaccn + jnp.where(seln, e, 0.0)
                return sacc, eaccp, eaccn

            sacc, eaccp, eaccn = lax.fori_loop(
                0, nvec4, p2, (zero16, zero16, zero16))
            s_below = jnp.sum(sacc)
            e_below = jnp.sum(eaccp)
            e_above = jnp.sum(eaccn)

            mid_p = (bp_eff.astype(jnp.float32) + 0.5) * _INV_SCALE - _BOUND
            mid_n = (bn_eff.astype(jnp.float32) + 0.5) * _INV_SCALE - _BOUND
            r_p = jnp.where(dp, k_p - cbp, 0.0)
            r_n = jnp.where(dn, k_n - can, 0.0)
            ep_mid = _lane0(jnp.exp(jnp.full((16,), (mid_p - 1.0) * _INVT)))
            en_mid = _lane0(jnp.exp(jnp.full((16,), (mid_n - 1.0) * _INVT)))
            s_pos = s_below + r_p * mid_p
            e_all = jnp.maximum(
                e_below + r_p * ep_mid + e_above + r_n * en_mid, 1e-37)
            ln_e = _lane0(_vln(jnp.full((16,), e_all)))
            loss_row = k_p * (_INVT + ln_e) - s_pos * _INVT
            nz_row = jnp.where(loss_row != 0.0, 1.0, 0.0)
            return loss_row, nz_row

        def pair_body(i, acc):
            loss_acc, nz_acc = acc
            r0 = base_row + 2 * i
            pltpu.make_async_copy(sim_hbm.at[r0], row0_v, sem0).wait()
            l0, n0 = process_row(row0_v, r0)

            @pl.when(i < rows_per_tile // 2 - 1)
            def _():
                pltpu.async_copy(sim_hbm.at[r0 + 2], row0_v, sem0)

            pltpu.make_async_copy(sim_hbm.at[r0 + 1], row1_v, sem1).wait()
            l1, n1 = process_row(row1_v, r0 + 1)

            @pl.when(i < rows_per_tile // 2 - 1)
            def _():
                pltpu.async_copy(sim_hbm.at[r0 + 3], row1_v, sem1)

            return loss_acc + l0 + l1, nz_acc + n0 + n1

        loss_sum, nz_sum = lax.fori_loop(0, rows_per_tile // 2, pair_body,
                                         (0.0, 0.0))
        out_vec = jnp.where(iota == 0, loss_sum,
                            jnp.where(iota == 1, nz_sum, 0.0))
        stage_v[...] = out_vec
        pltpu.sync_copy(stage_v, out_hbm.at[wid])

    return sc_sel


def kernel(old_feat, new_feat, target):
    B, D = old_feat.shape
    tc_rows = B - _SC_ROWS
    tgt = target.astype(jnp.int32)
    trow = tgt.astype(jnp.float32).reshape(B, 1)
    tcol = tgt.astype(jnp.float32).reshape(1, B)
    sim_tail = _tc_sim(old_feat, new_feat[tc_rows:])
    parts_sc = _make_sc(B, _SC_ROWS, tc_rows)(sim_tail, tgt)
    tc_loss, tc_nz = _tc_fused(old_feat, new_feat, trow, tcol, tc_rows)
    loss = tc_loss[0, 0] + jnp.sum(parts_sc[:, 0])
    nz = tc_nz[0, 0] + jnp.sum(parts_sc[:, 1])
    return loss / jnp.maximum(nz, 1.0)
